# baseline (speedup 1.0000x reference)
import functools

import jax
import jax.numpy as jnp
from jax import lax
from jax.experimental import pallas as pl
from jax.experimental.pallas import tpu as pltpu
from jax.experimental.pallas import tpu_sc as plsc

_C = 19
_NB = 16384
_NBT = 2 * _NB
_CHUNK = 8192
_NW = 32
_NH = 512 * 512
_SHARD = _C * _NH // _NW
_NCH = _SHARD // _CHUNK
_CPC = _NH // _CHUNK
assert _SHARD % _CHUNK == 0 and _NH % _CHUNK == 0


def _prep_body(o_ref, t_ref, comb_ref):
    o = o_ref[0]
    t = t_ref[0]
    m = t[0]
    lab = jnp.zeros(m.shape, jnp.int32)
    for c in range(1, _C):
        upd = t[c] > m
        m = jnp.where(upd, t[c], m)
        lab = jnp.where(upd, c, lab)
    rows = []
    for c in range(_C):
        isfg = lab == c
        err = jnp.where(isfg, jnp.abs(1.0 - o[c]), jnp.abs(o[c]))
        bits = lax.bitcast_convert_type(err, jnp.int32)
        comb = lax.shift_right_logical(bits, 17) + jnp.where(isfg, _NB, 0)
        rows.append(comb.astype(jnp.int16)[None])
    comb_ref[...] = jnp.concatenate(rows, axis=0)


def _prep_half(output, target, b0):
    B, C, H, W = output.shape
    bh = 16
    return pl.pallas_call(
        _prep_body,
        grid=(H // bh,),
        in_specs=[
            pl.BlockSpec((1, C, bh, W), lambda j, b0=b0: (b0, 0, j, 0)),
            pl.BlockSpec((1, C, bh, W), lambda j, b0=b0: (b0, 0, j, 0)),
        ],
        out_specs=pl.BlockSpec((C, bh, W), lambda j: (0, j, 0)),
        out_shape=jax.ShapeDtypeStruct((C, H, W), jnp.int16),
    )(output, target)


def _hist(comb_flat):
    mesh = plsc.VectorSubcoreMesh(core_axis_name="c", subcore_axis_name="s")

    @functools.partial(
        pl.kernel,
        out_type=jax.ShapeDtypeStruct((_NW, 2, _NBT), jnp.int32),
        mesh=mesh,
        compiler_params=pltpu.CompilerParams(needs_layout_passes=False),
        scratch_types=[
            pltpu.VMEM((2 * _CHUNK,), jnp.int16),
            pltpu.VMEM((2 * _NBT,), jnp.int32),
            pltpu.SemaphoreType.DMA,
            pltpu.SemaphoreType.DMA,
        ],
    )
    def hist_kernel(comb_hbm, tab_hbm, buf, tab, sem0, sem1):
        w = lax.axis_index("s") * 2 + lax.axis_index("c")
        base = w * _SHARD
        class_a = (w * _NCH) // _CPC
        sems = (sem0, sem1)

        def start(j):
            return pltpu.async_copy(
                comb_hbm.at[pl.ds(base + j * _CHUNK, _CHUNK)],
                buf.at[pl.ds((j % 2) * _CHUNK, _CHUNK)],
                sems[j % 2],
            )

        pending = start(0)
        zeros16 = jnp.zeros((16,), jnp.int32)

        def zero_body(i, carry):
            tab[pl.ds(i * 16, 16)] = zeros16
            return carry

        lax.fori_loop(0, (2 * _NBT) // 16, zero_body, 0)
        ones16 = jnp.ones((16,), jnp.int32)

        for j in range(_NCH):
            pending.wait()
            if j + 1 < _NCH:
                pending = start(j + 1)
            slot = (w * _NCH + j) // _CPC - class_a
            off = slot * _NBT
            bsel = j % 2

            def vec_body(i, carry, bsel=bsel, off=off):
                b0 = bsel * _CHUNK + i * 128
                for u in range(4):
                    pair = buf[pl.ds(b0 + u * 32, 32)]
                    ia, ib = plsc.unpack(
                        pair,
                        format=plsc.PackFormat.INTERLEAVED,
                        preferred_element_type=jnp.int32,
                    )
                    plsc.addupdate_scatter(tab, [ia + off], ones16)
                    plsc.addupdate_scatter(tab, [ib + off], ones16)
                return carry

            lax.fori_loop(0, _CHUNK // 128, vec_body, 0)
        pltpu.sync_copy(tab.at[pl.ds(0, _NBT)], tab_hbm.at[w, 0])
        pltpu.sync_copy(tab.at[pl.ds(_NBT, _NBT)], tab_hbm.at[w, 1])

    return hist_kernel(comb_flat)


def _merge_map():
    contrib = {c: [] for c in range(_C)}
    for w in range(_NW):
        classes = sorted({(w * _NCH + j) // _CPC for j in range(_NCH)})
        for s, c in enumerate(classes):
            contrib[c].append(w * 2 + s)
    return contrib


_CONTRIB = _merge_map()


def _finish_body(tab0_ref, tab1_ref, out_ref):
    T0 = tab0_ref[...].reshape(_NW * 2, _NBT).astype(jnp.float32)
    T1 = tab1_ref[...].reshape(_NW * 2, _NBT).astype(jnp.float32)
    rows = []
    for c in range(_C):
        acc = None
        for T in (T0, T1):
            for k in _CONTRIB[c]:
                piece = T[k:k + 1]
                acc = piece if acc is None else acc + piece
        rows.append(acc)
    tab = jnp.concatenate(rows, axis=0)
    C_ = _C
    f = tab[:, _NB:]
    n = tab[:, :_NB] + f

    blk = _NB // 128
    R = C_ * blk
    r128 = lax.broadcasted_iota(jnp.int32, (128, 128), 0)
    c128 = lax.broadcasted_iota(jnp.int32, (128, 128), 1)
    LT128 = (r128 >= c128).astype(jnp.float32)
    ONES128 = jnp.ones((128, 128), jnp.float32)
    rmod = lax.broadcasted_iota(jnp.int32, (R, 128), 0) % blk

    def suffix_minor(x):
        x2 = x.reshape(R, 128)
        sw = lax.dot_general(x2, LT128, (((1,), (0,)), ((), ())),
                             precision=lax.Precision.HIGHEST,
                             preferred_element_type=jnp.float32)
        t = lax.dot_general(x2, ONES128, (((1,), (0,)), ((), ())),
                            precision=lax.Precision.HIGHEST,
                            preferred_element_type=jnp.float32)
        z = t
        s = 1
        while s < blk:
            shifted = jnp.concatenate(
                [z[s:], jnp.zeros((s, 128), jnp.float32)], axis=0)
            z = z + jnp.where(rmod < blk - s, shifted, 0.0)
            s *= 2
        return (sw + z - t).reshape(C_, _NB)

    Sn = suffix_minor(n)
    Sf = suffix_minor(f)
    G = jnp.sum(f, axis=1, keepdims=True)

    def J(Nv, Fv):
        return 1.0 - (G - Fv) / jnp.maximum(G + Nv - Fv, 1.0)

    D = J(Sn, Sf) - J(Sn - n, Sf - f)
    k = lax.broadcasted_iota(jnp.int32, (C_, _NB), 1)
    midbits = lax.shift_left(k, 17) + 0x10000
    mid = lax.bitcast_convert_type(midbits, jnp.float32)
    mid = jnp.where(k < 0x3FC0, mid, 0.0)
    losses = jnp.sum(mid * D, axis=1)
    present = (G[:, 0] > 0.0).astype(jnp.float32)
    total = jnp.sum(losses * present) / jnp.maximum(jnp.sum(present), 1.0)
    out_ref[...] = total.reshape(1, 1)


def _finish(tab0, tab1):
    return pl.pallas_call(
        _finish_body,
        out_shape=jax.ShapeDtypeStruct((1, 1), jnp.float32),
    )(tab0, tab1)


def kernel(output, target):
    B, C, H, W = output.shape
    comb0 = _prep_half(output, target, 0)
    tab0 = _hist(comb0.reshape(C * H * W))
    comb1 = _prep_half(output, target, 1)
    tab1 = _hist(comb1.reshape(C * H * W))
    return _finish(tab0, tab1).reshape(())


# speedup vs baseline: 1.4375x; 1.4375x over previous
import functools

import jax
import jax.numpy as jnp
from jax import lax
from jax.experimental import pallas as pl
from jax.experimental.pallas import tpu as pltpu
from jax.experimental.pallas import tpu_sc as plsc

_C = 19
_NB = 16384
_NBT = 2 * _NB
_CHUNK = 8192
_NW = 32
_N = 2 * 512 * 512
_SHARD = _C * _N // _NW
_NCH = _SHARD // _CHUNK
_CPC = _N // _CHUNK
assert _SHARD % _CHUNK == 0 and _N % _CHUNK == 0


def _prep_body(o_ref, t_ref, comb_ref):
    o = o_ref[0]
    t = t_ref[0]
    m = t[0]
    lab = jnp.zeros(m.shape, jnp.int32)
    for c in range(1, _C):
        upd = t[c] > m
        m = jnp.where(upd, t[c], m)
        lab = jnp.where(upd, c, lab)
    rows = []
    for c in range(_C):
        isfg = lab == c
        err = jnp.where(isfg, jnp.abs(1.0 - o[c]), jnp.abs(o[c]))
        bits = lax.bitcast_convert_type(err, jnp.int32)
        comb = lax.shift_right_logical(bits, 17) + jnp.where(isfg, _NB, 0)
        rows.append(comb.astype(jnp.int16)[None])
    comb_ref[...] = jnp.concatenate(rows, axis=0)


def _prep(output, target):
    B, C, H, W = output.shape
    bh = 16
    return pl.pallas_call(
        _prep_body,
        grid=(B, H // bh),
        in_specs=[
            pl.BlockSpec((1, C, bh, W), lambda b, j: (b, 0, j, 0)),
            pl.BlockSpec((1, C, bh, W), lambda b, j: (b, 0, j, 0)),
        ],
        out_specs=pl.BlockSpec((C, bh, W), lambda b, j, H_b=H // bh: (0, b * H_b + j, 0)),
        out_shape=jax.ShapeDtypeStruct((C, B * H, W), jnp.int16),
    )(output, target)


def _hist(comb_flat):
    mesh = plsc.VectorSubcoreMesh(core_axis_name="c", subcore_axis_name="s")

    @functools.partial(
        pl.kernel,
        out_type=jax.ShapeDtypeStruct((_NW, 2, _NBT), jnp.int32),
        mesh=mesh,
        compiler_params=pltpu.CompilerParams(needs_layout_passes=False),
        scratch_types=[
            pltpu.VMEM((2 * _CHUNK,), jnp.int16),
            pltpu.VMEM((2 * _NBT,), jnp.int32),
            pltpu.SemaphoreType.DMA,
            pltpu.SemaphoreType.DMA,
        ],
    )
    def hist_kernel(comb_hbm, tab_hbm, buf, tab, sem0, sem1):
        w = lax.axis_index("s") * 2 + lax.axis_index("c")
        base = w * _SHARD
        class_a = (w * _NCH) // _CPC
        sems = (sem0, sem1)

        def start(j):
            return pltpu.async_copy(
                comb_hbm.at[pl.ds(base + j * _CHUNK, _CHUNK)],
                buf.at[pl.ds((j % 2) * _CHUNK, _CHUNK)],
                sems[j % 2],
            )

        pending = start(0)
        zeros16 = jnp.zeros((16,), jnp.int32)

        def zero_body(i, carry):
            tab[pl.ds(i * 16, 16)] = zeros16
            return carry

        lax.fori_loop(0, (2 * _NBT) // 16, zero_body, 0)
        ones16 = jnp.ones((16,), jnp.int32)

        for j in range(_NCH):
            pending.wait()
            if j + 1 < _NCH:
                pending = start(j + 1)
            slot = (w * _NCH + j) // _CPC - class_a
            off = slot * _NBT
            bsel = j % 2

            @plsc.parallel_loop(0, _CHUNK // 128, unroll=1)
            def vec_body(i, bsel=bsel, off=off):
                b0 = bsel * _CHUNK + i * 128
                for u in range(4):
                    pair = buf[pl.ds(b0 + u * 32, 32)]
                    ia, ib = plsc.unpack(
                        pair,
                        format=plsc.PackFormat.INTERLEAVED,
                        preferred_element_type=jnp.int32,
                    )
                    plsc.addupdate_scatter(tab, [ia + off], ones16)
                    plsc.addupdate_scatter(tab, [ib + off], ones16)
        pltpu.sync_copy(tab.at[pl.ds(0, _NBT)], tab_hbm.at[w, 0])
        pltpu.sync_copy(tab.at[pl.ds(_NBT, _NBT)], tab_hbm.at[w, 1])

    return hist_kernel(comb_flat)


def _merge_map():
    contrib = {c: [] for c in range(_C)}
    for w in range(_NW):
        classes = sorted({(w * _NCH + j) // _CPC for j in range(_NCH)})
        for s, c in enumerate(classes):
            contrib[c].append(w * 2 + s)
    return contrib


_CONTRIB = _merge_map()


def _finish_body(tab_ref, out_ref):
    T = tab_ref[...].reshape(_NW * 2, _NBT).astype(jnp.float32)
    rows = []
    for c in range(_C):
        acc = None
        for k in _CONTRIB[c]:
            piece = T[k:k + 1]
            acc = piece if acc is None else acc + piece
        rows.append(acc)
    tab = jnp.concatenate(rows, axis=0)
    C_ = _C
    f = tab[:, _NB:]
    n = tab[:, :_NB] + f

    blk = _NB // 128
    R = C_ * blk
    r128 = lax.broadcasted_iota(jnp.int32, (128, 128), 0)
    c128 = lax.broadcasted_iota(jnp.int32, (128, 128), 1)
    LT128 = (r128 >= c128).astype(jnp.float32)
    ONES128 = jnp.ones((128, 128), jnp.float32)
    rmod = lax.broadcasted_iota(jnp.int32, (R, 128), 0) % blk

    def suffix_minor(x):
        x2 = x.reshape(R, 128)
        sw = lax.dot_general(x2, LT128, (((1,), (0,)), ((), ())),
                             precision=lax.Precision.HIGHEST,
                             preferred_element_type=jnp.float32)
        t = lax.dot_general(x2, ONES128, (((1,), (0,)), ((), ())),
                            precision=lax.Precision.HIGHEST,
                            preferred_element_type=jnp.float32)
        z = t
        s = 1
        while s < blk:
            shifted = jnp.concatenate(
                [z[s:], jnp.zeros((s, 128), jnp.float32)], axis=0)
            z = z + jnp.where(rmod < blk - s, shifted, 0.0)
            s *= 2
        return (sw + z - t).reshape(C_, _NB)

    Sn = suffix_minor(n)
    Sf = suffix_minor(f)
    G = jnp.sum(f, axis=1, keepdims=True)

    def J(Nv, Fv):
        return 1.0 - (G - Fv) / jnp.maximum(G + Nv - Fv, 1.0)

    D = J(Sn, Sf) - J(Sn - n, Sf - f)
    k = lax.broadcasted_iota(jnp.int32, (C_, _NB), 1)
    midbits = lax.shift_left(k, 17) + 0x10000
    mid = lax.bitcast_convert_type(midbits, jnp.float32)
    mid = jnp.where(k < 0x3FC0, mid, 0.0)
    losses = jnp.sum(mid * D, axis=1)
    present = (G[:, 0] > 0.0).astype(jnp.float32)
    total = jnp.sum(losses * present) / jnp.maximum(jnp.sum(present), 1.0)
    out_ref[...] = total.reshape(1, 1)


def _finish(tab):
    return pl.pallas_call(
        _finish_body,
        out_shape=jax.ShapeDtypeStruct((1, 1), jnp.float32),
    )(tab)


def kernel(output, target):
    B, C, H, W = output.shape
    comb = _prep(output, target)
    tab = _hist(comb.reshape(C * B * H * W))
    return _finish(tab).reshape(())
